# baseline (speedup 1.0000x reference)
import jax
import jax.numpy as jnp
from jax import lax
from jax.experimental import pallas as pl

EMB = 256
HID = 256
NREL = 6
N = 512

_NEG = -9e15


def _net_kernel(feat_ref, adj_ref, wfc1_ref, bfc1_ref,
                w0_ref, as0_ref, ad0_ref, c0_ref,
                w1_ref, as1_ref, ad1_ref, c1_ref,
                out_ref):
    feat = feat_ref[0]
    adj = adj_ref[0]
    mask = adj > 0
    adj_bf = adj.astype(jnp.bfloat16)

    H = jnp.dot(feat, wfc1_ref[...], preferred_element_type=jnp.float32)
    H = jax.nn.relu(H + bfc1_ref[...])

    for w_ref, as_ref, ad_ref, c_ref in (
            (w0_ref, as0_ref, ad0_ref, c0_ref),
            (w1_ref, as1_ref, ad1_ref, c1_ref)):
        Wh = jnp.dot(H, w_ref[...], preferred_element_type=jnp.float32)
        s_src = jnp.dot(Wh, as_ref[...],
                        preferred_element_type=jnp.float32)
        s_dst = jnp.dot(Wh, ad_ref[...],
                        preferred_element_type=jnp.float32)

        rel = jnp.full((N, N), c_ref[0, 1].astype(jnp.bfloat16),
                       dtype=jnp.bfloat16)
        for r in range(2, NREL):
            rel = jnp.where(adj_bf == r,
                            c_ref[0, r].astype(jnp.bfloat16), rel)

        e = (s_src + s_dst.reshape(1, N)) + rel.astype(jnp.float32)
        e = jnp.maximum(e, 0.2 * e)
        e = jnp.where(mask, e, _NEG)
        m = jnp.max(e, axis=1, keepdims=True)
        p = jnp.exp(e - m)
        s = jnp.sum(p, axis=1, keepdims=True)
        inv = jnp.where(m > -8e15, 1.0 / s, 0.0)
        attn = p * inv

        out = jnp.dot(attn, Wh, preferred_element_type=jnp.float32)
        out = jnp.where(out > 0, out, jnp.exp(out) - 1.0)
        H = out + H

    out_ref[0, :, :HID] = H
    out_ref[0, :, HID:] = feat


@jax.jit
def kernel(utterance_features, semantic_adj, q_type, pos,
           W_fc1, b_fc1,
           W_gat0, a_src0, a_dst0, rel_bias0,
           W_gat1, a_src1, a_dst1, rel_bias1):
    del q_type, pos
    B = utterance_features.shape[0]

    row = lambda v: v.reshape(1, -1)
    col = lambda v: v.reshape(-1, 1)

    grid_spec = pl.GridSpec(
        grid=(B,),
        in_specs=[
            pl.BlockSpec((1, N, EMB), lambda b: (b, 0, 0)),
            pl.BlockSpec((1, N, N), lambda b: (b, 0, 0)),
            pl.BlockSpec((EMB, HID), lambda b: (0, 0)),
            pl.BlockSpec((1, HID), lambda b: (0, 0)),
            pl.BlockSpec((HID, HID), lambda b: (0, 0)),
            pl.BlockSpec((HID, 1), lambda b: (0, 0)),
            pl.BlockSpec((HID, 1), lambda b: (0, 0)),
            pl.BlockSpec((1, NREL), lambda b: (0, 0)),
            pl.BlockSpec((HID, HID), lambda b: (0, 0)),
            pl.BlockSpec((HID, 1), lambda b: (0, 0)),
            pl.BlockSpec((HID, 1), lambda b: (0, 0)),
            pl.BlockSpec((1, NREL), lambda b: (0, 0)),
        ],
        out_specs=pl.BlockSpec((1, N, HID + EMB), lambda b: (b, 0, 0)),
    )

    return pl.pallas_call(
        _net_kernel,
        grid_spec=grid_spec,
        out_shape=jax.ShapeDtypeStruct((B, N, HID + EMB), jnp.float32),
    )(utterance_features, semantic_adj,
      W_fc1, row(b_fc1),
      W_gat0, col(a_src0), col(a_dst0), row(rel_bias0),
      W_gat1, col(a_src1), col(a_dst1), row(rel_bias1))


# speedup vs baseline: 1.2684x; 1.2684x over previous
import jax
import jax.numpy as jnp
from jax import lax
from jax.experimental import pallas as pl

EMB = 256
HID = 256
NREL = 6
N = 512

_NEG = -9e15


def _net_kernel(feat_ref, adj_ref, wfc1_ref, bfc1_ref,
                w0_ref, as0_ref, ad0_ref, c0_ref,
                w1_ref, as1_ref, ad1_ref, c1_ref,
                out_ref):
    feat = feat_ref[0]
    adj = adj_ref[0]
    mask = adj > 0
    adj_bf = adj.astype(jnp.bfloat16)

    H = jnp.dot(feat, wfc1_ref[...], preferred_element_type=jnp.float32)
    H = jax.nn.relu(H + bfc1_ref[...])

    for w_ref, as_ref, ad_ref, c_ref in (
            (w0_ref, as0_ref, ad0_ref, c0_ref),
            (w1_ref, as1_ref, ad1_ref, c1_ref)):
        Wh = jnp.dot(H, w_ref[...], preferred_element_type=jnp.float32)
        s_src = jnp.sum(Wh * as_ref[...], axis=1, keepdims=True)
        s_dst = jnp.sum(Wh * ad_ref[...], axis=1, keepdims=True)

        rel = jnp.full((N, N), c_ref[0, 1].astype(jnp.bfloat16),
                       dtype=jnp.bfloat16)
        for r in range(2, NREL):
            rel = jnp.where(adj_bf == r,
                            c_ref[0, r].astype(jnp.bfloat16), rel)

        e = (s_src + s_dst.reshape(1, N)) + rel.astype(jnp.float32)
        e = jnp.maximum(e, 0.2 * e)
        e = jnp.where(mask, e, _NEG)
        m = jnp.max(e, axis=1, keepdims=True)
        p = jnp.exp(e - m)
        s = jnp.sum(p, axis=1, keepdims=True)
        inv = jnp.where(m > -8e15, 1.0 / s, 0.0)
        attn = p * inv

        out = jnp.dot(attn, Wh, preferred_element_type=jnp.float32)
        out = jnp.where(out > 0, out, jnp.exp(out) - 1.0)
        H = out + H

    out_ref[0, :, :HID] = H
    out_ref[0, :, HID:] = feat


@jax.jit
def kernel(utterance_features, semantic_adj, q_type, pos,
           W_fc1, b_fc1,
           W_gat0, a_src0, a_dst0, rel_bias0,
           W_gat1, a_src1, a_dst1, rel_bias1):
    del q_type, pos
    B = utterance_features.shape[0]

    row = lambda v: v.reshape(1, -1)

    grid_spec = pl.GridSpec(
        grid=(B,),
        in_specs=[
            pl.BlockSpec((1, N, EMB), lambda b: (b, 0, 0)),
            pl.BlockSpec((1, N, N), lambda b: (b, 0, 0)),
            pl.BlockSpec((EMB, HID), lambda b: (0, 0)),
            pl.BlockSpec((1, HID), lambda b: (0, 0)),
            pl.BlockSpec((HID, HID), lambda b: (0, 0)),
            pl.BlockSpec((1, HID), lambda b: (0, 0)),
            pl.BlockSpec((1, HID), lambda b: (0, 0)),
            pl.BlockSpec((1, NREL), lambda b: (0, 0)),
            pl.BlockSpec((HID, HID), lambda b: (0, 0)),
            pl.BlockSpec((1, HID), lambda b: (0, 0)),
            pl.BlockSpec((1, HID), lambda b: (0, 0)),
            pl.BlockSpec((1, NREL), lambda b: (0, 0)),
        ],
        out_specs=pl.BlockSpec((1, N, HID + EMB), lambda b: (b, 0, 0)),
    )

    return pl.pallas_call(
        _net_kernel,
        grid_spec=grid_spec,
        out_shape=jax.ShapeDtypeStruct((B, N, HID + EMB), jnp.float32),
    )(utterance_features, semantic_adj,
      W_fc1, row(b_fc1),
      W_gat0, row(a_src0), row(a_dst0), row(rel_bias0),
      W_gat1, row(a_src1), row(a_dst1), row(rel_bias1))


# speedup vs baseline: 1.3805x; 1.0884x over previous
import jax
import jax.numpy as jnp
from jax import lax
from jax.experimental import pallas as pl

EMB = 256
HID = 256
NREL = 6
N = 512

_NEG = -9e15


def _net_kernel(feat_ref, adj_ref, wfc1_ref, bfc1_ref,
                w0_ref, as0_ref, ad0_ref, c0_ref,
                w1_ref, as1_ref, ad1_ref, c1_ref,
                out_ref):
    feat = feat_ref[0]
    adj = adj_ref[0]
    mask = adj > 0
    adj_bf = adj.astype(jnp.bfloat16)

    H = jnp.dot(feat, wfc1_ref[...], preferred_element_type=jnp.float32)
    H = jax.nn.relu(H + bfc1_ref[...])

    for w_ref, as_ref, ad_ref, c_ref in (
            (w0_ref, as0_ref, ad0_ref, c0_ref),
            (w1_ref, as1_ref, ad1_ref, c1_ref)):
        Wh = jnp.dot(H, w_ref[...], preferred_element_type=jnp.float32)
        s_src = jnp.sum(Wh * as_ref[...], axis=1, keepdims=True)
        s_dst = jnp.sum(Wh * ad_ref[...], axis=1, keepdims=True)

        rel = jnp.full((N, N), c_ref[0, 1].astype(jnp.bfloat16),
                       dtype=jnp.bfloat16)
        for r in range(2, NREL):
            rel = jnp.where(adj_bf == r,
                            c_ref[0, r].astype(jnp.bfloat16), rel)

        e = (s_src + s_dst.reshape(1, N)) + rel.astype(jnp.float32)
        e = jnp.maximum(e, 0.2 * e)
        e = jnp.where(mask, e, _NEG)
        m = jnp.max(e, axis=1, keepdims=True)
        p = jnp.exp(e - m)
        s = jnp.sum(p, axis=1, keepdims=True)
        inv = jnp.where(m > -8e15, 1.0 / s, 0.0)

        out = jnp.dot(p, Wh, preferred_element_type=jnp.float32) * inv
        out = jnp.where(out > 0, out, jnp.exp(out) - 1.0)
        H = out + H

    out_ref[0, :, :HID] = H
    out_ref[0, :, HID:] = feat


@jax.jit
def kernel(utterance_features, semantic_adj, q_type, pos,
           W_fc1, b_fc1,
           W_gat0, a_src0, a_dst0, rel_bias0,
           W_gat1, a_src1, a_dst1, rel_bias1):
    del q_type, pos
    B = utterance_features.shape[0]

    row = lambda v: v.reshape(1, -1)

    grid_spec = pl.GridSpec(
        grid=(B,),
        in_specs=[
            pl.BlockSpec((1, N, EMB), lambda b: (b, 0, 0)),
            pl.BlockSpec((1, N, N), lambda b: (b, 0, 0)),
            pl.BlockSpec((EMB, HID), lambda b: (0, 0)),
            pl.BlockSpec((1, HID), lambda b: (0, 0)),
            pl.BlockSpec((HID, HID), lambda b: (0, 0)),
            pl.BlockSpec((1, HID), lambda b: (0, 0)),
            pl.BlockSpec((1, HID), lambda b: (0, 0)),
            pl.BlockSpec((1, NREL), lambda b: (0, 0)),
            pl.BlockSpec((HID, HID), lambda b: (0, 0)),
            pl.BlockSpec((1, HID), lambda b: (0, 0)),
            pl.BlockSpec((1, HID), lambda b: (0, 0)),
            pl.BlockSpec((1, NREL), lambda b: (0, 0)),
        ],
        out_specs=pl.BlockSpec((1, N, HID + EMB), lambda b: (b, 0, 0)),
    )

    return pl.pallas_call(
        _net_kernel,
        grid_spec=grid_spec,
        out_shape=jax.ShapeDtypeStruct((B, N, HID + EMB), jnp.float32),
    )(utterance_features, semantic_adj,
      W_fc1, row(b_fc1),
      W_gat0, row(a_src0), row(a_dst0), row(rel_bias0),
      W_gat1, row(a_src1), row(a_dst1), row(rel_bias1))
